# baseline (speedup 1.0000x reference)
import functools

import jax
import jax.numpy as jnp
from jax import lax
from jax.experimental import pallas as pl
from jax.experimental.pallas import tpu as pltpu
from jax.experimental.pallas import tpu_sc as plsc

D_MODEL = 128
SEQ = 20
NUM_CORES = 2
NUM_SUBCORES = 16
NUM_WORKERS = NUM_CORES * NUM_SUBCORES
ECHUNK = 64
NBUF = 5


@jax.jit
def _sc_gather(time_t, pe):
    n_batch = time_t.shape[1]
    b_per_w = n_batch // NUM_WORKERS
    cpj = b_per_w // ECHUNK
    cpj_shift = cpj.bit_length() - 1
    assert cpj == 1 << cpj_shift
    n_chunks = SEQ * cpj
    n_super = n_chunks // NBUF
    mesh = plsc.VectorSubcoreMesh(core_axis_name="c", subcore_axis_name="s")

    @functools.partial(
        pl.kernel,
        mesh=mesh,
        out_type=jax.ShapeDtypeStruct((SEQ, n_batch, D_MODEL), jnp.float32),
        scratch_types=[
            pltpu.VMEM((SEQ, b_per_w), jnp.int32),
            pltpu.VMEM((2, NBUF, ECHUNK, D_MODEL), jnp.float32),
            pltpu.VMEM_SHARED((367, D_MODEL), jnp.float32),
            pltpu.SemaphoreType.DMA,
            pltpu.SemaphoreType.DMA,
        ],
    )
    def k(time_hbm, pe_hbm, out_hbm, idx_v, rows_v, pe_sp, gsem, ssem):
        sid = lax.axis_index("s")
        wid = sid * NUM_CORES + lax.axis_index("c")
        base_b = wid * b_per_w

        @pl.when(sid == 0)
        def _():
            pltpu.sync_copy(pe_hbm, pe_sp)

        pltpu.sync_copy(time_hbm.at[:, pl.ds(base_b, b_per_w)], idx_v)
        plsc.subcore_barrier()

        def chunk_coords(c):
            j = lax.shift_right_logical(c, cpj_shift)
            blk = lax.bitwise_and(c, cpj - 1)
            return j, blk * ECHUNK

        def fire_gathers(s, half):
            for b in range(NBUF):
                j, off = chunk_coords(s * NBUF + b)
                pltpu.async_copy(
                    pe_sp.at[idx_v.at[j, pl.ds(off, ECHUNK)]],
                    rows_v.at[half, b],
                    gsem,
                )

        def drain_gathers(s, half):
            for b in range(NBUF):
                j, off = chunk_coords(s * NBUF + b)
                pltpu.make_async_copy(
                    pe_sp.at[idx_v.at[j, pl.ds(off, ECHUNK)]],
                    rows_v.at[half, b],
                    gsem,
                ).wait()

        def fire_stores(s, half):
            for b in range(NBUF):
                j, off = chunk_coords(s * NBUF + b)
                pltpu.async_copy(
                    rows_v.at[half, b],
                    out_hbm.at[j, pl.ds(base_b + off, ECHUNK)],
                    ssem,
                )

        def drain_stores(s, half):
            for b in range(NBUF):
                j, off = chunk_coords(s * NBUF + b)
                pltpu.make_async_copy(
                    rows_v.at[half, b],
                    out_hbm.at[j, pl.ds(base_b + off, ECHUNK)],
                    ssem,
                ).wait()

        fire_gathers(0, 0)

        def body(s, carry):
            half = lax.rem(s, 2)
            drain_gathers(s, half)

            @pl.when(s >= 1)
            def _():
                drain_stores(s - 1, 1 - half)

            @pl.when(s + 1 < n_super)
            def _():
                fire_gathers(s + 1, 1 - half)

            fire_stores(s, half)
            return carry

        lax.fori_loop(0, n_super, body, 0)
        drain_stores(n_super - 1, lax.rem(n_super - 1, 2))

    return k(time_t, pe)


def kernel(time, pe):
    time_t = time.astype(jnp.int32).T
    out_t = _sc_gather(time_t, pe)
    return out_t.transpose(1, 0, 2)


# speedup vs baseline: 1.0417x; 1.0417x over previous
import functools

import jax
import jax.numpy as jnp
from jax import lax
from jax.experimental import pallas as pl
from jax.experimental.pallas import tpu as pltpu
from jax.experimental.pallas import tpu_sc as plsc

D_MODEL = 128
SEQ = 20
NUM_CORES = 2
NUM_SUBCORES = 16
NUM_WORKERS = NUM_CORES * NUM_SUBCORES
ECHUNK = 128
NBUF = 2


@jax.jit
def _sc_gather(time_t, pe):
    n_batch = time_t.shape[1]
    b_per_w = n_batch // NUM_WORKERS
    cpj = b_per_w // ECHUNK
    cpj_shift = cpj.bit_length() - 1
    assert cpj == 1 << cpj_shift
    n_chunks = SEQ * cpj
    n_super = n_chunks // NBUF
    mesh = plsc.VectorSubcoreMesh(core_axis_name="c", subcore_axis_name="s")

    @functools.partial(
        pl.kernel,
        mesh=mesh,
        out_type=jax.ShapeDtypeStruct((SEQ, n_batch, D_MODEL), jnp.float32),
        scratch_types=[
            pltpu.VMEM((SEQ, b_per_w), jnp.int32),
            pltpu.VMEM((3, NBUF, ECHUNK, D_MODEL), jnp.float32),
            pltpu.VMEM_SHARED((367, D_MODEL), jnp.float32),
            pltpu.SemaphoreType.DMA,
            pltpu.SemaphoreType.DMA,
        ],
    )
    def k(time_hbm, pe_hbm, out_hbm, idx_v, rows_v, pe_sp, gsem, ssem):
        sid = lax.axis_index("s")
        wid = sid * NUM_CORES + lax.axis_index("c")
        base_b = wid * b_per_w

        @pl.when(sid == 0)
        def _():
            pltpu.sync_copy(pe_hbm, pe_sp)

        pltpu.sync_copy(time_hbm.at[:, pl.ds(base_b, b_per_w)], idx_v)
        plsc.subcore_barrier()

        def chunk_coords(c):
            j = lax.shift_right_logical(c, cpj_shift)
            blk = lax.bitwise_and(c, cpj - 1)
            return j, blk * ECHUNK

        def fire_gathers(s, half):
            for b in range(NBUF):
                j, off = chunk_coords(s * NBUF + b)
                pltpu.async_copy(
                    pe_sp.at[idx_v.at[j, pl.ds(off, ECHUNK)]],
                    rows_v.at[half, b],
                    gsem,
                )

        def drain_gathers(s, half):
            for b in range(NBUF):
                j, off = chunk_coords(s * NBUF + b)
                pltpu.make_async_copy(
                    pe_sp.at[idx_v.at[j, pl.ds(off, ECHUNK)]],
                    rows_v.at[half, b],
                    gsem,
                ).wait()

        def fire_stores(s, half):
            for b in range(NBUF):
                j, off = chunk_coords(s * NBUF + b)
                pltpu.async_copy(
                    rows_v.at[half, b],
                    out_hbm.at[j, pl.ds(base_b + off, ECHUNK)],
                    ssem,
                )

        def drain_stores(s, half):
            for b in range(NBUF):
                j, off = chunk_coords(s * NBUF + b)
                pltpu.make_async_copy(
                    rows_v.at[half, b],
                    out_hbm.at[j, pl.ds(base_b + off, ECHUNK)],
                    ssem,
                ).wait()

        fire_gathers(0, 0)
        fire_gathers(1, 1)

        def body(s, carry):
            r = lax.rem(s, 3)
            drain_gathers(s, r)
            fire_stores(s, r)

            @pl.when(s >= 1)
            def _():
                drain_stores(s - 1, lax.rem(s - 1, 3))

            @pl.when(s + 2 < n_super)
            def _():
                fire_gathers(s + 2, lax.rem(s + 2, 3))

            return carry

        lax.fori_loop(0, n_super, body, 0)
        drain_stores(n_super - 1, lax.rem(n_super - 1, 3))

    return k(time_t, pe)


def kernel(time, pe):
    time_t = time.astype(jnp.int32).T
    out_t = _sc_gather(time_t, pe)
    return out_t.transpose(1, 0, 2)


# speedup vs baseline: 1.0528x; 1.0106x over previous
import functools

import jax
import jax.numpy as jnp
from jax import lax
from jax.experimental import pallas as pl
from jax.experimental.pallas import tpu as pltpu
from jax.experimental.pallas import tpu_sc as plsc

D_MODEL = 128
SEQ = 20
NUM_CORES = 2
NUM_SUBCORES = 16
NUM_WORKERS = NUM_CORES * NUM_SUBCORES
ECHUNK = 128
NBUF = 2


@jax.jit
def _sc_gather(time_t, pe):
    n_batch = time_t.shape[1]
    b_per_w = n_batch // NUM_WORKERS
    cpj = b_per_w // ECHUNK
    cpj_shift = cpj.bit_length() - 1
    assert cpj == 1 << cpj_shift
    n_chunks = SEQ * cpj
    n_super = n_chunks // NBUF
    mesh = plsc.VectorSubcoreMesh(core_axis_name="c", subcore_axis_name="s")

    @functools.partial(
        pl.kernel,
        mesh=mesh,
        out_type=jax.ShapeDtypeStruct((SEQ, n_batch, D_MODEL), jnp.float32),
        scratch_types=[
            pltpu.VMEM((SEQ, b_per_w), jnp.int32),
            pltpu.VMEM((3, NBUF, ECHUNK, D_MODEL), jnp.float32),
            pltpu.VMEM_SHARED((367, D_MODEL), jnp.float32),
            pltpu.SemaphoreType.DMA,
            pltpu.SemaphoreType.DMA,
            pltpu.SemaphoreType.DMA,
        ],
    )
    def k(time_hbm, pe_hbm, out_hbm, idx_v, rows_v, pe_sp, gsem, ssem, psem):
        sid = lax.axis_index("s")
        wid = sid * NUM_CORES + lax.axis_index("c")
        base_b = wid * b_per_w

        def pe_copy_main():
            return pltpu.make_async_copy(
                pe_hbm.at[pl.ds(sid * 24, 24)],
                pe_sp.at[pl.ds(sid * 24, 24)],
                psem,
            )

        def pe_copy_tail():
            return pltpu.make_async_copy(
                pe_hbm.at[pl.ds(360, 7)], pe_sp.at[pl.ds(360, 7)], psem
            )

        @pl.when(sid < 15)
        def _():
            pe_copy_main().start()

        @pl.when(sid == 15)
        def _():
            pe_copy_tail().start()

        pltpu.sync_copy(time_hbm.at[:, pl.ds(base_b, b_per_w)], idx_v)

        @pl.when(sid < 15)
        def _():
            pe_copy_main().wait()

        @pl.when(sid == 15)
        def _():
            pe_copy_tail().wait()

        plsc.subcore_barrier()

        def chunk_coords(c):
            j = lax.shift_right_logical(c, cpj_shift)
            blk = lax.bitwise_and(c, cpj - 1)
            return j, blk * ECHUNK

        def fire_gathers(s, half):
            for b in range(NBUF):
                j, off = chunk_coords(s * NBUF + b)
                pltpu.async_copy(
                    pe_sp.at[idx_v.at[j, pl.ds(off, ECHUNK)]],
                    rows_v.at[half, b],
                    gsem,
                )

        def drain_gathers(s, half):
            for b in range(NBUF):
                j, off = chunk_coords(s * NBUF + b)
                pltpu.make_async_copy(
                    pe_sp.at[idx_v.at[j, pl.ds(off, ECHUNK)]],
                    rows_v.at[half, b],
                    gsem,
                ).wait()

        def fire_stores(s, half):
            for b in range(NBUF):
                j, off = chunk_coords(s * NBUF + b)
                pltpu.async_copy(
                    rows_v.at[half, b],
                    out_hbm.at[j, pl.ds(base_b + off, ECHUNK)],
                    ssem,
                )

        def drain_stores(s, half):
            for b in range(NBUF):
                j, off = chunk_coords(s * NBUF + b)
                pltpu.make_async_copy(
                    rows_v.at[half, b],
                    out_hbm.at[j, pl.ds(base_b + off, ECHUNK)],
                    ssem,
                ).wait()

        fire_gathers(0, 0)
        fire_gathers(1, 1)

        def body(s, carry):
            r = lax.rem(s, 3)
            drain_gathers(s, r)
            fire_stores(s, r)

            @pl.when(s >= 1)
            def _():
                drain_stores(s - 1, lax.rem(s - 1, 3))

            @pl.when(s + 2 < n_super)
            def _():
                fire_gathers(s + 2, lax.rem(s + 2, 3))

            return carry

        lax.fori_loop(0, n_super, body, 0)
        drain_stores(n_super - 1, lax.rem(n_super - 1, 3))

    return k(time_t, pe)


def kernel(time, pe):
    time_t = time.astype(jnp.int32).T
    out_t = _sc_gather(time_t, pe)
    return out_t.transpose(1, 0, 2)


# speedup vs baseline: 1.0593x; 1.0062x over previous
import functools

import jax
import jax.numpy as jnp
from jax import lax
from jax.experimental import pallas as pl
from jax.experimental.pallas import tpu as pltpu
from jax.experimental.pallas import tpu_sc as plsc

D_MODEL = 128
SEQ = 20
NUM_CORES = 2
NUM_SUBCORES = 16
NUM_WORKERS = NUM_CORES * NUM_SUBCORES
ECHUNK = 128
NBUF = 1


@jax.jit
def _sc_gather(time_t, pe):
    n_batch = time_t.shape[1]
    b_per_w = n_batch // NUM_WORKERS
    cpj = b_per_w // ECHUNK
    cpj_shift = cpj.bit_length() - 1
    assert cpj == 1 << cpj_shift
    n_chunks = SEQ * cpj
    n_super = n_chunks // NBUF
    mesh = plsc.VectorSubcoreMesh(core_axis_name="c", subcore_axis_name="s")

    @functools.partial(
        pl.kernel,
        mesh=mesh,
        out_type=jax.ShapeDtypeStruct((SEQ, n_batch, D_MODEL), jnp.float32),
        scratch_types=[
            pltpu.VMEM((SEQ, b_per_w), jnp.int32),
            pltpu.VMEM((3, NBUF, ECHUNK, D_MODEL), jnp.float32),
            pltpu.VMEM_SHARED((367, D_MODEL), jnp.float32),
            pltpu.SemaphoreType.DMA,
            pltpu.SemaphoreType.DMA,
            pltpu.SemaphoreType.DMA,
        ],
    )
    def k(time_hbm, pe_hbm, out_hbm, idx_v, rows_v, pe_sp, gsem, ssem, psem):
        sid = lax.axis_index("s")
        wid = sid * NUM_CORES + lax.axis_index("c")
        base_b = wid * b_per_w

        def pe_copy_main():
            return pltpu.make_async_copy(
                pe_hbm.at[pl.ds(sid * 24, 24)],
                pe_sp.at[pl.ds(sid * 24, 24)],
                psem,
            )

        def pe_copy_tail():
            return pltpu.make_async_copy(
                pe_hbm.at[pl.ds(360, 7)], pe_sp.at[pl.ds(360, 7)], psem
            )

        @pl.when(sid < 15)
        def _():
            pe_copy_main().start()

        @pl.when(sid == 15)
        def _():
            pe_copy_tail().start()

        pltpu.sync_copy(time_hbm.at[:, pl.ds(base_b, b_per_w)], idx_v)

        @pl.when(sid < 15)
        def _():
            pe_copy_main().wait()

        @pl.when(sid == 15)
        def _():
            pe_copy_tail().wait()

        plsc.subcore_barrier()

        def chunk_coords(c):
            j = lax.shift_right_logical(c, cpj_shift)
            blk = lax.bitwise_and(c, cpj - 1)
            return j, blk * ECHUNK

        def fire_gathers(s, half):
            for b in range(NBUF):
                j, off = chunk_coords(s * NBUF + b)
                pltpu.async_copy(
                    pe_sp.at[idx_v.at[j, pl.ds(off, ECHUNK)]],
                    rows_v.at[half, b],
                    gsem,
                )

        def drain_gathers(s, half):
            for b in range(NBUF):
                j, off = chunk_coords(s * NBUF + b)
                pltpu.make_async_copy(
                    pe_sp.at[idx_v.at[j, pl.ds(off, ECHUNK)]],
                    rows_v.at[half, b],
                    gsem,
                ).wait()

        def fire_stores(s, half):
            for b in range(NBUF):
                j, off = chunk_coords(s * NBUF + b)
                pltpu.async_copy(
                    rows_v.at[half, b],
                    out_hbm.at[j, pl.ds(base_b + off, ECHUNK)],
                    ssem,
                )

        def drain_stores(s, half):
            for b in range(NBUF):
                j, off = chunk_coords(s * NBUF + b)
                pltpu.make_async_copy(
                    rows_v.at[half, b],
                    out_hbm.at[j, pl.ds(base_b + off, ECHUNK)],
                    ssem,
                ).wait()

        fire_gathers(0, 0)
        fire_gathers(1, 1)

        def body(s, carry):
            r = lax.rem(s, 3)
            drain_gathers(s, r)
            fire_stores(s, r)

            @pl.when(s >= 1)
            def _():
                drain_stores(s - 1, lax.rem(s - 1, 3))

            @pl.when(s + 2 < n_super)
            def _():
                fire_gathers(s + 2, lax.rem(s + 2, 3))

            return carry

        lax.fori_loop(0, n_super, body, 0)
        drain_stores(n_super - 1, lax.rem(n_super - 1, 3))

    return k(time_t, pe)


def kernel(time, pe):
    time_t = time.astype(jnp.int32).T
    out_t = _sc_gather(time_t, pe)
    return out_t.transpose(1, 0, 2)
